# Initial kernel scaffold; baseline (speedup 1.0000x reference)
import jax
import jax.numpy as jnp
from jax.experimental import pallas as pl


def kernel(x, emb, w_out):
    raise NotImplementedError("write your pallas kernel here")


# speedup vs baseline: 7.6524x; 7.6524x over previous
import functools

import jax
import jax.numpy as jnp
from jax import lax
from jax.experimental import pallas as pl
from jax.experimental.pallas import tpu as pltpu
from jax.experimental.pallas import tpu_sc as plsc

_NC = 2
_NS = 16
_LN = 16
_NW = _NC * _NS


def _table_body(emb_ref, w_ref, out_ref):
    logits = jnp.dot(emb_ref[...], w_ref[...],
                     preferred_element_type=jnp.float32)
    m = jnp.max(logits, axis=1, keepdims=True)
    lse = m + jnp.log(jnp.sum(jnp.exp(logits - m), axis=1, keepdims=True))
    out_ref[...] = lse - logits


def _build_nll_table(emb, w_out):
    v = emb.shape[0]
    return pl.pallas_call(
        _table_body,
        out_shape=jax.ShapeDtypeStruct((v, v), jnp.float32),
    )(emb, w_out)


def _sc_gather_partials(table_flat, xt, xl, vocab, n_valid):
    total = xt.shape[0]
    c = total // _NW
    gw = 128
    ng = c // gw
    mesh = plsc.VectorSubcoreMesh(
        core_axis_name="c", subcore_axis_name="s",
        num_cores=_NC, num_subcores=_NS)

    @functools.partial(
        pl.kernel,
        out_type=jax.ShapeDtypeStruct((_NW, _LN), jnp.float32),
        mesh=mesh,
        scratch_types=[
            pltpu.VMEM((c,), jnp.int32),
            pltpu.VMEM((c,), jnp.int32),
            pltpu.VMEM((ng, gw), jnp.int32),
            pltpu.VMEM((ng, gw), jnp.float32),
            pltpu.VMEM((_LN,), jnp.float32),
            pltpu.SemaphoreType.DMA,
        ],
    )
    def k(table_hbm, xt_hbm, xl_hbm, out_hbm,
          xt_v, xl_v, idx_v, val_v, vec_v, sem):
        cid = lax.axis_index("c")
        sid = lax.axis_index("s")
        wid = sid * _NC + cid
        base = wid * c
        pltpu.sync_copy(xt_hbm.at[pl.ds(base, c)], xt_v)
        pltpu.sync_copy(xl_hbm.at[pl.ds(base, c)], xl_v)
        for j in range(ng):
            for i in range(gw // _LN):
                sl = pl.ds(j * gw + i * _LN, _LN)
                idx_v[j, pl.ds(i * _LN, _LN)] = xt_v[sl] * vocab + xl_v[sl]
        copies = [pltpu.async_copy(table_hbm.at[idx_v.at[j]], val_v.at[j], sem)
                  for j in range(ng)]
        for cp in copies:
            cp.wait()
        lanes = lax.iota(jnp.int32, _LN)
        acc = jnp.zeros((_LN,), jnp.float32)
        for j in range(ng):
            for i in range(gw // _LN):
                g = base + j * gw + i * _LN + lanes
                vals = val_v[j, pl.ds(i * _LN, _LN)]
                acc = acc + jnp.where(g < n_valid, vals, 0.0)
        vec_v[...] = acc
        pltpu.sync_copy(vec_v, out_hbm.at[wid])

    return k(table_flat, xt, xl)


def kernel(x, emb, w_out):
    vocab = emb.shape[0]
    b, s = x.shape
    n_valid = b * (s - 1)

    x = x.astype(jnp.int32)
    xt = x[:, :-1].reshape(-1)
    xl = x[:, 1:].reshape(-1)
    chunk = _NW * _LN
    padded = ((n_valid + chunk - 1) // chunk) * chunk
    pad = padded - n_valid
    xt = jnp.pad(xt, (0, pad))
    xl = jnp.pad(xl, (0, pad))

    table = _build_nll_table(emb, w_out)
    partials = _sc_gather_partials(table.reshape(-1), xt, xl, vocab, n_valid)
    return jnp.sum(partials) / n_valid


# speedup vs baseline: 7.8923x; 1.0314x over previous
import functools

import jax
import jax.numpy as jnp
from jax import lax
from jax.experimental import pallas as pl
from jax.experimental.pallas import tpu as pltpu
from jax.experimental.pallas import tpu_sc as plsc

_NC = 2
_NS = 16
_LN = 16
_NW = _NC * _NS


def _table_body(emb_ref, w_ref, out_ref):
    logits = jnp.dot(emb_ref[...], w_ref[...],
                     preferred_element_type=jnp.float32)
    m = jnp.max(logits, axis=1, keepdims=True)
    lse = m + jnp.log(jnp.sum(jnp.exp(logits - m), axis=1, keepdims=True))
    out_ref[...] = lse - logits


def _build_nll_table(emb, w_out):
    v = emb.shape[0]
    return pl.pallas_call(
        _table_body,
        out_shape=jax.ShapeDtypeStruct((v, v), jnp.float32),
    )(emb, w_out)


def _sc_gather_partials(table_flat, xf, vocab, s_sz):
    total = xf.shape[0]
    c = total // _NW
    gw = 128
    ng = c // gw
    mesh = plsc.VectorSubcoreMesh(
        core_axis_name="c", subcore_axis_name="s",
        num_cores=_NC, num_subcores=_NS)

    @functools.partial(
        pl.kernel,
        out_type=jax.ShapeDtypeStruct((_NW, _LN), jnp.float32),
        mesh=mesh,
        scratch_types=[
            pltpu.VMEM((c + _LN,), jnp.int32),
            pltpu.VMEM((ng, gw), jnp.int32),
            pltpu.VMEM((ng, gw), jnp.float32),
            pltpu.VMEM((_LN,), jnp.float32),
            pltpu.SemaphoreType.DMA,
        ],
    )
    def k(table_hbm, x_hbm, out_hbm, xv, idx_v, val_v, vec_v, sem):
        cid = lax.axis_index("c")
        sid = lax.axis_index("s")
        wid = sid * _NC + cid
        base = wid * c

        @pl.when(wid == _NW - 1)
        def _():
            pltpu.sync_copy(x_hbm.at[pl.ds(base, c)], xv.at[pl.ds(0, c)])

        @pl.when(wid != _NW - 1)
        def _():
            pltpu.sync_copy(x_hbm.at[pl.ds(base, c + 8)],
                            xv.at[pl.ds(0, c + 8)])

        lanes = lax.iota(jnp.int32, _LN)
        shift = jnp.where(lanes == _LN - 1, 0, lanes + 1)
        zeros = jnp.zeros((_LN,), jnp.int32)
        last = lanes == _LN - 1
        for j in range(ng):
            for i in range(gw // _LN):
                o = j * gw + i * _LN
                t = xv[pl.ds(o, _LN)]
                t_next = xv[pl.ds(o + _LN, _LN)]
                lbl = jnp.where(
                    last,
                    t_next.at[zeros].get(mode='promise_in_bounds'),
                    t.at[shift].get(mode='promise_in_bounds'))
                p = base + o + lanes
                valid = lax.rem(p, s_sz) != s_sz - 1
                idx_v[j, pl.ds(i * _LN, _LN)] = jnp.where(
                    valid, t * vocab + lbl, 0)
        copies = [pltpu.async_copy(table_hbm.at[idx_v.at[j]], val_v.at[j],
                                   sem) for j in range(ng)]
        for cp in copies:
            cp.wait()
        acc = jnp.zeros((_LN,), jnp.float32)
        for j in range(ng):
            for i in range(gw // _LN):
                p = base + j * gw + i * _LN + lanes
                vals = val_v[j, pl.ds(i * _LN, _LN)]
                acc = acc + jnp.where(lax.rem(p, s_sz) != s_sz - 1,
                                      vals, 0.0)
        vec_v[...] = acc
        pltpu.sync_copy(vec_v, out_hbm.at[wid])

    return k(table_flat, xf)


def kernel(x, emb, w_out):
    vocab = emb.shape[0]
    b, s = x.shape
    n_valid = b * (s - 1)
    table = _build_nll_table(emb, w_out)
    partials = _sc_gather_partials(table.reshape(-1),
                                   x.astype(jnp.int32).reshape(-1), vocab, s)
    return jnp.sum(partials) / n_valid


# speedup vs baseline: 9.1974x; 1.1654x over previous
import functools

import jax
import jax.numpy as jnp
from jax import lax
from jax.experimental import pallas as pl
from jax.experimental.pallas import tpu as pltpu
from jax.experimental.pallas import tpu_sc as plsc

_NC = 2
_NS = 16
_LN = 16
_NW = _NC * _NS


def _table_body(emb_ref, w_ref, out_ref):
    logits = jnp.dot(emb_ref[...], w_ref[...],
                     preferred_element_type=jnp.float32)
    m = jnp.max(logits, axis=1, keepdims=True)
    lse = m + jnp.log(jnp.sum(jnp.exp(logits - m), axis=1, keepdims=True))
    nll = lse - logits
    v = nll.shape[0]
    padded = jnp.pad(nll, ((0, 0), (0, 1024 - v)))
    out_ref[...] = padded.reshape(-1)


def _build_nll_table(emb, w_out):
    v = emb.shape[0]
    return pl.pallas_call(
        _table_body,
        out_shape=jax.ShapeDtypeStruct((v * 1024,), jnp.float32),
    )(emb, w_out)


def _sc_gather_partials(table_flat, xf, vocab, s_sz):
    total = xf.shape[0]
    c = total // _NW
    gw = 128
    ng = c // gw
    mesh = plsc.VectorSubcoreMesh(
        core_axis_name="c", subcore_axis_name="s",
        num_cores=_NC, num_subcores=_NS)

    @functools.partial(
        pl.kernel,
        out_type=jax.ShapeDtypeStruct((_NW, _LN), jnp.float32),
        mesh=mesh,
        scratch_types=[
            pltpu.VMEM((c + _LN,), jnp.int32),
            pltpu.VMEM((ng, gw), jnp.int32),
            pltpu.VMEM((ng, gw), jnp.float32),
            pltpu.VMEM((_LN,), jnp.float32),
            pltpu.SemaphoreType.DMA,
        ],
    )
    def k(table_hbm, x_hbm, out_hbm, xv, idx_v, val_v, vec_v, sem):
        cid = lax.axis_index("c")
        sid = lax.axis_index("s")
        wid = sid * _NC + cid
        base = wid * c

        @pl.when(wid == _NW - 1)
        def _():
            pltpu.sync_copy(x_hbm.at[pl.ds(base, c)], xv.at[pl.ds(0, c)])

        @pl.when(wid != _NW - 1)
        def _():
            pltpu.sync_copy(x_hbm.at[pl.ds(base, c + 8)],
                            xv.at[pl.ds(0, c + 8)])

        lanes = lax.iota(jnp.int32, _LN)
        shift = jnp.where(lanes == _LN - 1, 0, lanes + 1)
        zeros = jnp.zeros((_LN,), jnp.int32)
        last = lanes == _LN - 1
        for j in range(ng):
            for i in range(gw // _LN):
                o = j * gw + i * _LN
                t = xv[pl.ds(o, _LN)]
                t_next = xv[pl.ds(o + _LN, _LN)]
                lbl = jnp.where(
                    last,
                    t_next.at[zeros].get(mode='promise_in_bounds'),
                    t.at[shift].get(mode='promise_in_bounds'))
                p = base + o + lanes
                valid = lax.rem(p, s_sz) != s_sz - 1
                idx_v[j, pl.ds(i * _LN, _LN)] = jnp.where(
                    valid, t * 1024 + lbl, 0)
        copies = [pltpu.async_copy(table_hbm.at[idx_v.at[j]], val_v.at[j],
                                   sem) for j in range(ng)]
        for cp in copies:
            cp.wait()
        acc = jnp.zeros((_LN,), jnp.float32)
        for j in range(ng):
            for i in range(gw // _LN):
                p = base + j * gw + i * _LN + lanes
                vals = val_v[j, pl.ds(i * _LN, _LN)]
                acc = acc + jnp.where(lax.rem(p, s_sz) != s_sz - 1,
                                      vals, 0.0)
        vec_v[...] = acc
        pltpu.sync_copy(vec_v, out_hbm.at[wid])

    return k(table_flat, xf)


def kernel(x, emb, w_out):
    vocab = emb.shape[0]
    b, s = x.shape
    n_valid = b * (s - 1)
    table = _build_nll_table(emb, w_out)
    partials = _sc_gather_partials(table,
                                   x.astype(jnp.int32).reshape(-1), vocab, s)
    return jnp.sum(partials) / n_valid
